# Initial kernel scaffold; baseline (speedup 1.0000x reference)
import jax
import jax.numpy as jnp
from jax.experimental import pallas as pl


def kernel(features, tables):
    raise NotImplementedError("write your pallas kernel here")


# speedup vs baseline: 1.1789x; 1.1789x over previous
import functools

import jax
import jax.numpy as jnp
from jax import lax
from jax.experimental import pallas as pl
from jax.experimental.pallas import tpu as pltpu
from jax.experimental.pallas import tpu_sc as plsc

N_FIELDS = 26
N_VOCAB = 100000
D = 32
B = 16384

NC = 2
NS = 16
NW = NC * NS
LANES = 16

SAMPLES_PER_W = B // NW
CHUNK = 64
ROWS = CHUNK * N_FIELDS
IDX_BLK = 128
N_BLK = ROWS // IDX_BLK
N_CHUNKS = SAMPLES_PER_W // CHUNK


def _body(feat_hbm, off_hbm, table_hbm, out_hbm,
          feat_v, off_v, idx_v, rows_v, out_v, sem):
    wid = lax.axis_index("s") * NC + lax.axis_index("c")
    base = wid * SAMPLES_PER_W

    pltpu.sync_copy(off_hbm, off_v)

    def chunk_body(g, carry):
        row0 = base + g * CHUNK
        pltpu.sync_copy(feat_hbm.at[pl.ds(row0 * N_FIELDS, ROWS)], feat_v)

        def idx_body(j, c):
            jb = j // 8
            jl = j % 8
            v = feat_v[pl.ds(j * LANES, LANES)] + off_v[pl.ds(j * LANES, LANES)]
            idx_v[jb, pl.ds(jl * LANES, LANES)] = v
            return c
        lax.fori_loop(0, ROWS // LANES, idx_body, 0)

        cps = [
            pltpu.make_async_copy(
                table_hbm.at[idx_v.at[jb]],
                rows_v.at[pl.ds(jb * IDX_BLK, IDX_BLK)],
                sem,
            )
            for jb in range(N_BLK)
        ]
        for cp in cps:
            cp.start()
        for cp in cps:
            cp.wait()

        def sum_body(s, c):
            r0 = s * N_FIELDS
            a0 = rows_v[r0, pl.ds(0, LANES)]
            a1 = rows_v[r0, pl.ds(LANES, LANES)]
            for k in range(1, N_FIELDS):
                a0 = a0 + rows_v[r0 + k, pl.ds(0, LANES)]
                a1 = a1 + rows_v[r0 + k, pl.ds(LANES, LANES)]
            out_v[s, pl.ds(0, LANES)] = a0
            out_v[s, pl.ds(LANES, LANES)] = a1
            return c
        lax.fori_loop(0, CHUNK, sum_body, 0)

        pltpu.sync_copy(out_v, out_hbm.at[pl.ds(row0, CHUNK)])
        return carry

    lax.fori_loop(0, N_CHUNKS, chunk_body, 0)


@jax.jit
def _run(feat_flat, offsets, table_flat):
    mesh = plsc.VectorSubcoreMesh(core_axis_name="c", subcore_axis_name="s")
    f = functools.partial(
        pl.kernel,
        out_type=jax.ShapeDtypeStruct((B, D), jnp.float32),
        mesh=mesh,
        scratch_types=[
            pltpu.VMEM((ROWS,), jnp.int32),
            pltpu.VMEM((ROWS,), jnp.int32),
            pltpu.VMEM((N_BLK, IDX_BLK), jnp.int32),
            pltpu.VMEM((ROWS, D), jnp.float32),
            pltpu.VMEM((CHUNK, D), jnp.float32),
            pltpu.SemaphoreType.DMA,
        ],
        compiler_params=pltpu.CompilerParams(use_tc_tiling_on_sc=False),
    )(_body)
    return f(feat_flat, offsets, table_flat)


def kernel(features, tables):
    feat_flat = features.astype(jnp.int32).reshape(B * N_FIELDS)
    table_flat = tables.reshape(N_FIELDS * N_VOCAB, D)
    offsets = jnp.tile(jnp.arange(N_FIELDS, dtype=jnp.int32) * N_VOCAB, CHUNK)
    return _run(feat_flat, offsets, table_flat)


# speedup vs baseline: 3.0968x; 2.6269x over previous
import functools

import jax
import jax.numpy as jnp
from jax import lax
from jax.experimental import pallas as pl
from jax.experimental.pallas import tpu as pltpu
from jax.experimental.pallas import tpu_sc as plsc

N_FIELDS = 26
N_VOCAB = 100000
D = 32
B = 16384

NC = 2
NS = 16
LANES = 16

FEAT_CHUNK = 8192


def _body(featT_hbm, tabT_hbm, out_hbm, feat_v, tab_v, acc_v, sem_t):
    d = lax.axis_index("c") * NS + lax.axis_index("s")

    zero = jnp.zeros((LANES,), jnp.float32)

    def zero_body(j, carry):
        acc_v[pl.ds(j * LANES, LANES)] = zero
        return carry
    lax.fori_loop(0, B // LANES, zero_body, 0)

    def field_body(i, carry):
        cp = pltpu.make_async_copy(tabT_hbm.at[i, d, :], tab_v, sem_t)
        cp.start()
        cp.wait()

        def half_body(fb, c2):
            pltpu.sync_copy(featT_hbm.at[i, pl.ds(fb * FEAT_CHUNK, FEAT_CHUNK)],
                            feat_v)

            def samp_body(j, c3):
                v = feat_v[pl.ds(j * LANES, LANES)]
                g = plsc.load_gather(tab_v, [v])
                base = fb * FEAT_CHUNK + j * LANES
                acc_v[pl.ds(base, LANES)] = acc_v[pl.ds(base, LANES)] + g
                return c3
            lax.fori_loop(0, FEAT_CHUNK // LANES, samp_body, 0)
            return c2
        lax.fori_loop(0, B // FEAT_CHUNK, half_body, 0)
        return carry

    lax.fori_loop(0, N_FIELDS, field_body, 0)
    pltpu.sync_copy(acc_v, out_hbm.at[d])


@jax.jit
def _run(featT, tabT):
    mesh = plsc.VectorSubcoreMesh(core_axis_name="c", subcore_axis_name="s")
    f = functools.partial(
        pl.kernel,
        out_type=jax.ShapeDtypeStruct((D, B), jnp.float32),
        mesh=mesh,
        scratch_types=[
            pltpu.VMEM((FEAT_CHUNK,), jnp.int32),
            pltpu.VMEM((N_VOCAB,), jnp.float32),
            pltpu.VMEM((B,), jnp.float32),
            pltpu.SemaphoreType.DMA,
        ],
        compiler_params=pltpu.CompilerParams(
            use_tc_tiling_on_sc=True, needs_layout_passes=False
        ),
    )(_body)
    return f(featT, tabT)


def kernel(features, tables):
    featT = features.astype(jnp.int32).T
    tabT = jnp.transpose(tables, (0, 2, 1))
    outT = _run(featT, tabT)
    return outT.T


# speedup vs baseline: 4.1457x; 1.3387x over previous
import functools

import jax
import jax.numpy as jnp
from jax import lax
from jax.experimental import pallas as pl
from jax.experimental.pallas import tpu as pltpu
from jax.experimental.pallas import tpu_sc as plsc

N_FIELDS = 26
N_VOCAB = 100000
D = 32
B = 16384

NC = 2
NS = 16
LANES = 16

FEAT_CHUNK = 8192
UNROLL = 8


def _body(featT_hbm, tabT_hbm, out_hbm, feat_v, tab_v, acc_v, sem_t):
    d = lax.axis_index("c") * NS + lax.axis_index("s")

    zero = jnp.zeros((LANES,), jnp.float32)

    def zero_body(j, carry):
        acc_v[pl.ds(j * LANES, LANES)] = zero
        return carry
    lax.fori_loop(0, B // LANES, zero_body, 0)

    def field_body(i, carry):
        cp = pltpu.make_async_copy(tabT_hbm.at[i, d, :], tab_v, sem_t)
        cp.start()
        cp.wait()

        def half_body(fb, c2):
            pltpu.sync_copy(featT_hbm.at[i, pl.ds(fb * FEAT_CHUNK, FEAT_CHUNK)],
                            feat_v)

            def samp_body(j, c3):
                for u in range(UNROLL):
                    off = j * LANES * UNROLL + u * LANES
                    v = feat_v[pl.ds(off, LANES)]
                    g = plsc.load_gather(tab_v, [v])
                    base = fb * FEAT_CHUNK + off
                    acc_v[pl.ds(base, LANES)] = acc_v[pl.ds(base, LANES)] + g
                return c3
            lax.fori_loop(0, FEAT_CHUNK // (LANES * UNROLL), samp_body, 0)
            return c2
        lax.fori_loop(0, B // FEAT_CHUNK, half_body, 0)
        return carry

    lax.fori_loop(0, N_FIELDS, field_body, 0)
    pltpu.sync_copy(acc_v, out_hbm.at[d])


@jax.jit
def _run(featT, tabT):
    mesh = plsc.VectorSubcoreMesh(core_axis_name="c", subcore_axis_name="s")
    f = functools.partial(
        pl.kernel,
        out_type=jax.ShapeDtypeStruct((D, B), jnp.float32),
        mesh=mesh,
        scratch_types=[
            pltpu.VMEM((FEAT_CHUNK,), jnp.int32),
            pltpu.VMEM((N_VOCAB,), jnp.float32),
            pltpu.VMEM((B,), jnp.float32),
            pltpu.SemaphoreType.DMA,
        ],
        compiler_params=pltpu.CompilerParams(
            use_tc_tiling_on_sc=True, needs_layout_passes=False
        ),
    )(_body)
    return f(featT, tabT)


def kernel(features, tables):
    featT = features.astype(jnp.int32).T
    tabT = jnp.transpose(tables, (0, 2, 1))
    outT = _run(featT, tabT)
    return outT.T
